# baseline (speedup 1.0000x reference)
import jax
import jax.numpy as jnp
from jax.experimental import pallas as pl


def _body(x_ref, t_ref, o_ref, *, P, M, D, scale):
    c = pl.program_id(1)
    k0 = c * P
    main = t_ref[pl.ds(k0, P), :]
    prev = t_ref[pl.ds(jnp.maximum(k0 - 1, 0), 1), :]
    nxt = t_ref[pl.ds(jnp.minimum(k0 + P, M - 1), 1), :]
    left = jnp.concatenate([prev, main], axis=0)
    right = jnp.concatenate([main, nxt], axis=0)
    fw = jnp.concatenate([left, right], axis=1)

    j = k0 + jax.lax.broadcasted_iota(jnp.int32, (P, 1), 0)
    spe = (2 * j).astype(jnp.float32) * scale
    spo = (2 * j + 1).astype(jnp.float32) * scale
    we = spe - jnp.floor(spe)
    wo = spo - jnp.floor(spo)
    w = jnp.concatenate(
        [jnp.broadcast_to(we, (P, D)), jnp.broadcast_to(wo, (P, D))], axis=1
    )
    emb = (1.0 - w) * fw[:P] + w * fw[1:]
    o_ref[0] = x_ref[0] + emb


def kernel(x, pos_table):
    B, S, D = x.shape
    M = pos_table.shape[0]
    P = 256
    NP = S // 2
    scale = (M - 1) / (S - 1)
    xw = x.reshape(B, NP, 2 * D)

    import functools
    body = functools.partial(_body, P=P, M=M, D=D, scale=scale)
    out = pl.pallas_call(
        body,
        grid=(B, NP // P),
        in_specs=[
            pl.BlockSpec((1, P, 2 * D), lambda b, c: (b, c, 0)),
            pl.BlockSpec((M, D), lambda b, c: (0, 0)),
        ],
        out_specs=pl.BlockSpec((1, P, 2 * D), lambda b, c: (b, c, 0)),
        out_shape=jax.ShapeDtypeStruct((B, NP, 2 * D), x.dtype),
    )(xw, pos_table)
    return out.reshape(B, S, D)


# speedup vs baseline: 2.4572x; 2.4572x over previous
import functools

import jax
import jax.numpy as jnp
from jax.experimental import pallas as pl
from jax.experimental.pallas import tpu as pltpu


def _body(x_ref, t_ref, o_ref, *, P, M, D, scale):
    c = pl.program_id(1)
    k0 = c * P
    main = t_ref[pl.ds(k0, P), :]
    prev = t_ref[pl.ds(jnp.maximum(k0 - 1, 0), 1), :]
    nxt = t_ref[pl.ds(jnp.minimum(k0 + P, M - 1), 1), :]
    row = jax.lax.broadcasted_iota(jnp.int32, (P, 1), 0)
    a = jnp.where(row == 0, prev, pltpu.roll(main, 1, 0))
    cc = jnp.where(row == P - 1, nxt, pltpu.roll(main, P - 1, 0))

    j = k0 + row
    spe = (2 * j).astype(jnp.float32) * scale
    spo = (2 * j + 1).astype(jnp.float32) * scale
    we = spe - jnp.floor(spe)
    wo = spo - jnp.floor(spo)
    emb_even = (1.0 - we) * a + we * main
    emb_odd = (1.0 - wo) * main + wo * cc
    emb = jnp.stack([emb_even, emb_odd], axis=1).reshape(2 * P, D)
    o_ref[0] = x_ref[0] + emb


def kernel(x, pos_table):
    B, S, D = x.shape
    M = pos_table.shape[0]
    P = 256
    scale = (M - 1) / (S - 1)

    body = functools.partial(_body, P=P, M=M, D=D, scale=scale)
    return pl.pallas_call(
        body,
        grid=(B, M // P),
        in_specs=[
            pl.BlockSpec((1, 2 * P, D), lambda b, c: (b, c, 0)),
            pl.BlockSpec((M, D), lambda b, c: (0, 0)),
        ],
        out_specs=pl.BlockSpec((1, 2 * P, D), lambda b, c: (b, c, 0)),
        out_shape=jax.ShapeDtypeStruct((B, S, D), x.dtype),
    )(x, pos_table)


# speedup vs baseline: 2.9303x; 1.1925x over previous
import functools

import jax
import jax.numpy as jnp
from jax.experimental import pallas as pl
from jax.experimental.pallas import tpu as pltpu


def _body(x_ref, t_ref, o_ref, emb_ref, *, P, M, D, scale):
    c = pl.program_id(0)
    b = pl.program_id(1)

    @pl.when(b == 0)
    def _build_emb():
        k0 = c * P
        main = t_ref[pl.ds(k0, P), :]
        prev = t_ref[pl.ds(jnp.maximum(k0 - 1, 0), 1), :]
        nxt = t_ref[pl.ds(jnp.minimum(k0 + P, M - 1), 1), :]
        row = jax.lax.broadcasted_iota(jnp.int32, (P, 1), 0)
        a = jnp.where(row == 0, prev, pltpu.roll(main, 1, 0))
        cc = jnp.where(row == P - 1, nxt, pltpu.roll(main, P - 1, 0))

        j = k0 + row
        spe = (2 * j).astype(jnp.float32) * scale
        spo = (2 * j + 1).astype(jnp.float32) * scale
        we = spe - jnp.floor(spe)
        wo = spo - jnp.floor(spo)
        emb_even = (1.0 - we) * a + we * main
        emb_odd = (1.0 - wo) * main + wo * cc
        emb_ref[...] = jnp.stack([emb_even, emb_odd], axis=1).reshape(2 * P, D)

    o_ref[0] = x_ref[0] + emb_ref[...]


def kernel(x, pos_table):
    B, S, D = x.shape
    M = pos_table.shape[0]
    P = 256
    scale = (M - 1) / (S - 1)

    body = functools.partial(_body, P=P, M=M, D=D, scale=scale)
    return pl.pallas_call(
        body,
        grid=(M // P, B),
        in_specs=[
            pl.BlockSpec((1, 2 * P, D), lambda c, b: (b, c, 0)),
            pl.BlockSpec((M, D), lambda c, b: (0, 0)),
        ],
        out_specs=pl.BlockSpec((1, 2 * P, D), lambda c, b: (b, c, 0)),
        out_shape=jax.ShapeDtypeStruct((B, S, D), x.dtype),
        scratch_shapes=[pltpu.VMEM((2 * P, D), jnp.float32)],
    )(x, pos_table)


# speedup vs baseline: 3.1644x; 1.0799x over previous
import functools

import jax
import jax.numpy as jnp
from jax.experimental import pallas as pl
from jax.experimental.pallas import tpu as pltpu


def _body(x_ref, t_ref, o_ref, emb_ref, *, P, M, D, scale):
    c = pl.program_id(0)
    b = pl.program_id(1)

    @pl.when(b == 0)
    def _build_emb():
        k0 = c * P
        main = t_ref[pl.ds(k0, P), :]
        prev = t_ref[pl.ds(jnp.maximum(k0 - 1, 0), 1), :]
        nxt = t_ref[pl.ds(jnp.minimum(k0 + P, M - 1), 1), :]
        row = jax.lax.broadcasted_iota(jnp.int32, (P, 1), 0)
        a = jnp.where(row == 0, prev, pltpu.roll(main, 1, 0))
        cc = jnp.where(row == P - 1, nxt, pltpu.roll(main, P - 1, 0))

        j = k0 + row
        spe = (2 * j).astype(jnp.float32) * scale
        spo = (2 * j + 1).astype(jnp.float32) * scale
        we = spe - jnp.floor(spe)
        wo = spo - jnp.floor(spo)
        emb_even = (1.0 - we) * a + we * main
        emb_odd = (1.0 - wo) * main + wo * cc
        emb_ref[...] = jnp.stack([emb_even, emb_odd], axis=1).reshape(2 * P, D)

    o_ref[0] = x_ref[0] + emb_ref[...]


def kernel(x, pos_table):
    B, S, D = x.shape
    M = pos_table.shape[0]
    P = 512
    scale = (M - 1) / (S - 1)

    body = functools.partial(_body, P=P, M=M, D=D, scale=scale)
    return pl.pallas_call(
        body,
        grid=(M // P, B),
        in_specs=[
            pl.BlockSpec((1, 2 * P, D), lambda c, b: (b, c, 0)),
            pl.BlockSpec((M, D), lambda c, b: (0, 0)),
        ],
        out_specs=pl.BlockSpec((1, 2 * P, D), lambda c, b: (b, c, 0)),
        out_shape=jax.ShapeDtypeStruct((B, S, D), x.dtype),
        scratch_shapes=[pltpu.VMEM((2 * P, D), jnp.float32)],
    )(x, pos_table)


# speedup vs baseline: 3.2940x; 1.0410x over previous
import functools

import jax
import jax.numpy as jnp
from jax.experimental import pallas as pl
from jax.experimental.pallas import tpu as pltpu


def _body(x_ref, t_ref, o_ref, emb_ref, *, P, M, D, scale):
    c = pl.program_id(0)
    b = pl.program_id(1)

    @pl.when(b == 0)
    def _build_emb():
        k0 = c * P
        main = t_ref[pl.ds(k0, P), :]
        prev = t_ref[pl.ds(jnp.maximum(k0 - 1, 0), 1), :]
        nxt = t_ref[pl.ds(jnp.minimum(k0 + P, M - 1), 1), :]
        row = jax.lax.broadcasted_iota(jnp.int32, (P, 1), 0)
        a = jnp.where(row == 0, prev, pltpu.roll(main, 1, 0))
        cc = jnp.where(row == P - 1, nxt, pltpu.roll(main, P - 1, 0))

        j = k0 + row
        spe = (2 * j).astype(jnp.float32) * scale
        spo = (2 * j + 1).astype(jnp.float32) * scale
        we = spe - jnp.floor(spe)
        wo = spo - jnp.floor(spo)
        emb_even = (1.0 - we) * a + we * main
        emb_odd = (1.0 - wo) * main + wo * cc
        emb_ref[...] = jnp.stack([emb_even, emb_odd], axis=1).reshape(2 * P, D)

    o_ref[0] = x_ref[0] + emb_ref[...]


def kernel(x, pos_table):
    B, S, D = x.shape
    M = pos_table.shape[0]
    P = 1024
    scale = (M - 1) / (S - 1)

    body = functools.partial(_body, P=P, M=M, D=D, scale=scale)
    return pl.pallas_call(
        body,
        grid=(M // P, B),
        in_specs=[
            pl.BlockSpec((1, 2 * P, D), lambda c, b: (b, c, 0)),
            pl.BlockSpec((M, D), lambda c, b: (0, 0)),
        ],
        out_specs=pl.BlockSpec((1, 2 * P, D), lambda c, b: (b, c, 0)),
        out_shape=jax.ShapeDtypeStruct((B, S, D), x.dtype),
        scratch_shapes=[pltpu.VMEM((2 * P, D), jnp.float32)],
    )(x, pos_table)
